# baseline (speedup 1.0000x reference)
import functools

import jax
import jax.numpy as jnp
from jax import lax
from jax.experimental import pallas as pl
from jax.experimental.pallas import tpu as pltpu


def _fused_body(x_ref, o_ref, *, k):
    xb = x_ref[0]
    _, hw = xb.shape

    energy = jnp.sum(xb, axis=0, keepdims=True)

    bits = lax.bitcast_convert_type(energy, jnp.uint32)
    sign = (bits >> 31).astype(jnp.uint32)
    key = bits ^ jnp.where(sign == 1, jnp.uint32(0xFFFFFFFF), jnp.uint32(0x80000000))

    kk = jnp.int32(k)

    def val_step(i, t):
        bit = jnp.uint32(31) - i.astype(jnp.uint32)
        cand = t | (jnp.uint32(1) << bit)
        cnt = jnp.sum((key >= cand).astype(jnp.int32))
        return jnp.where(cnt >= kk, cand, t)

    t = lax.fori_loop(0, 32, val_step, jnp.uint32(0))

    count_gt = jnp.sum((key > t).astype(jnp.int32))
    extra = kk - count_gt
    eq = key == t
    idx = lax.broadcasted_iota(jnp.int32, (1, hw), 1)

    def idx_step(i, j):
        bit = jnp.int32(10) - i
        cand = j | (jnp.int32(1) << bit)
        cnt = jnp.sum((eq & (idx < cand)).astype(jnp.int32))
        return jnp.where(cnt <= extra, cand, j)

    jmax = lax.fori_loop(0, 11, idx_step, jnp.int32(0))

    keep = (key > t) | (eq & (idx < jmax))
    o_ref[0] = jnp.where(keep, xb, jnp.float32(0.0))


def kernel(x):
    B, C, H, W = x.shape
    HW = H * W
    k = int(HW * 0.5)
    xf = x.reshape(B, C, HW)

    out = pl.pallas_call(
        functools.partial(_fused_body, k=k),
        grid=(B,),
        in_specs=[pl.BlockSpec((1, C, HW), lambda b: (b, 0, 0))],
        out_specs=pl.BlockSpec((1, C, HW), lambda b: (b, 0, 0)),
        out_shape=jax.ShapeDtypeStruct((B, C, HW), jnp.float32),
        compiler_params=pltpu.CompilerParams(
            dimension_semantics=("arbitrary",),
        ),
    )(xf)
    return out.reshape(B, C, H, W)


# speedup vs baseline: 1.5010x; 1.5010x over previous
import functools

import jax
import jax.numpy as jnp
from jax import lax
from jax.experimental import pallas as pl
from jax.experimental.pallas import tpu as pltpu


def _fused_body(x_ref, o_ref, *, k):
    xb = x_ref[0]
    _, hw = xb.shape

    energy = jnp.sum(xb, axis=0, keepdims=True)

    bits = lax.bitcast_convert_type(energy, jnp.uint32)
    sign = (bits >> 31).astype(jnp.uint32)
    key_row = bits ^ jnp.where(
        sign == 1, jnp.uint32(0xFFFFFFFF), jnp.uint32(0x80000000)
    )
    key_col = jnp.reshape(key_row, (hw, 1))

    i_row = lax.broadcasted_iota(jnp.int32, (1, hw), 1)
    j_col = lax.broadcasted_iota(jnp.int32, (hw, 1), 0)
    beats = (key_col > key_row) | ((key_col == key_row) & (j_col < i_row))
    cnt = jnp.sum(beats.astype(jnp.int32), axis=0, keepdims=True)

    keep = cnt < jnp.int32(k)
    o_ref[0] = jnp.where(keep, xb, jnp.float32(0.0))


def kernel(x):
    B, C, H, W = x.shape
    HW = H * W
    k = int(HW * 0.5)
    xf = x.reshape(B, C, HW)

    out = pl.pallas_call(
        functools.partial(_fused_body, k=k),
        grid=(B,),
        in_specs=[pl.BlockSpec((1, C, HW), lambda b: (b, 0, 0))],
        out_specs=pl.BlockSpec((1, C, HW), lambda b: (b, 0, 0)),
        out_shape=jax.ShapeDtypeStruct((B, C, HW), jnp.float32),
        compiler_params=pltpu.CompilerParams(
            dimension_semantics=("arbitrary",),
        ),
    )(xf)
    return out.reshape(B, C, H, W)
